# baseline (speedup 1.0000x reference)
import functools

import jax
import jax.numpy as jnp
from jax import lax
from jax.experimental import pallas as pl
from jax.experimental.pallas import tpu as pltpu
from jax.experimental.pallas import tpu_sc as plsc

N = 10000
E = 320000
NC = 2
NS = 16
NW = NC * NS
EPT = E // NW
RPT = 624
TAIL = N - NS * RPT

_MESH = plsc.VectorSubcoreMesh(core_axis_name="c", subcore_axis_name="s")
_SC_PARAMS = pltpu.CompilerParams(use_tc_tiling_on_sc=False)


def _make_spmm(D, K):
    NB = EPT // K

    def body(h_hbm, src_hbm, dst_hbm, zrow_hbm, out_hbm,
             accum, htab, sidx, didx, rows, gsem):
        c = lax.axis_index("c")
        s = lax.axis_index("s")
        tid = s * NC + c
        r0 = pl.multiple_of(s * RPT, 8)
        pltpu.sync_copy(zrow_hbm, accum.at[pl.ds(r0, RPT)])
        pltpu.sync_copy(h_hbm.at[pl.ds(r0, RPT)], htab.at[pl.ds(r0, RPT)])

        @pl.when(s == NS - 1)
        def _():
            pltpu.sync_copy(zrow_hbm.at[pl.ds(0, TAIL)],
                            accum.at[pl.ds(NS * RPT, TAIL)])
            pltpu.sync_copy(h_hbm.at[pl.ds(NS * RPT, TAIL)],
                            htab.at[pl.ds(NS * RPT, TAIL)])

        pltpu.sync_copy(src_hbm.at[tid], sidx)
        pltpu.sync_copy(dst_hbm.at[tid], didx)
        plsc.subcore_barrier()

        def step(i, carry):
            b = lax.rem(i, 2)
            pltpu.make_async_copy(htab.at[sidx.at[i]], rows.at[b], gsem).wait()

            @pl.when(i + 1 < NB)
            def _():
                pltpu.async_copy(htab.at[sidx.at[i + 1]],
                                 rows.at[1 - b], gsem)

            pltpu.sync_copy(rows.at[b], accum.at[didx.at[i]], add=True)
            return carry

        pltpu.async_copy(htab.at[sidx.at[0]], rows.at[0], gsem)
        lax.fori_loop(0, NB, step, 0)
        plsc.subcore_barrier()
        pltpu.sync_copy(accum.at[pl.ds(r0, RPT)],
                        out_hbm.at[c, pl.ds(r0, RPT)])

        @pl.when(s == NS - 1)
        def _():
            pltpu.sync_copy(accum.at[pl.ds(NS * RPT, TAIL)],
                            out_hbm.at[c, pl.ds(NS * RPT, TAIL)])

    return pl.kernel(
        body,
        out_type=jax.ShapeDtypeStruct((NC, N, D), jnp.float32),
        mesh=_MESH,
        compiler_params=_SC_PARAMS,
        scratch_types=[
            pltpu.VMEM_SHARED((N, D), jnp.float32),
            pltpu.VMEM_SHARED((N, D), jnp.float32),
            pltpu.VMEM((NB, K), jnp.int32),
            pltpu.VMEM((NB, K), jnp.int32),
            pltpu.VMEM((2, K, D), jnp.float32),
            pltpu.SemaphoreType.DMA,
        ],
    )


def _make_spmm2(D, K):
    NB = EPT // K

    def body(ha_hbm, hb_hbm, src_hbm, dst_hbm, zrow_hbm, outa_hbm, outb_hbm,
             accum, htab, sidx, didx, rows, gsem):
        c = lax.axis_index("c")
        s = lax.axis_index("s")
        tid = s * NC + c
        r0 = pl.multiple_of(s * RPT, 8)
        pltpu.sync_copy(src_hbm.at[tid], sidx)
        pltpu.sync_copy(dst_hbm.at[tid], didx)

        def run_pass(h_hbm, out_hbm):
            pltpu.sync_copy(zrow_hbm, accum.at[pl.ds(r0, RPT)])
            pltpu.sync_copy(h_hbm.at[pl.ds(r0, RPT)], htab.at[pl.ds(r0, RPT)])

            @pl.when(s == NS - 1)
            def _():
                pltpu.sync_copy(zrow_hbm.at[pl.ds(0, TAIL)],
                                accum.at[pl.ds(NS * RPT, TAIL)])
                pltpu.sync_copy(h_hbm.at[pl.ds(NS * RPT, TAIL)],
                                htab.at[pl.ds(NS * RPT, TAIL)])

            plsc.subcore_barrier()

            def step(i, carry):
                b = lax.rem(i, 2)
                pltpu.make_async_copy(htab.at[sidx.at[i]], rows.at[b],
                                      gsem).wait()

                @pl.when(i + 1 < NB)
                def _():
                    pltpu.async_copy(htab.at[sidx.at[i + 1]],
                                     rows.at[1 - b], gsem)

                pltpu.sync_copy(rows.at[b], accum.at[didx.at[i]], add=True)
                return carry

            pltpu.async_copy(htab.at[sidx.at[0]], rows.at[0], gsem)
            lax.fori_loop(0, NB, step, 0)
            plsc.subcore_barrier()
            pltpu.sync_copy(accum.at[pl.ds(r0, RPT)],
                            out_hbm.at[c, pl.ds(r0, RPT)])

            @pl.when(s == NS - 1)
            def _():
                pltpu.sync_copy(accum.at[pl.ds(NS * RPT, TAIL)],
                                out_hbm.at[c, pl.ds(NS * RPT, TAIL)])

        run_pass(ha_hbm, outa_hbm)
        run_pass(hb_hbm, outb_hbm)

    return pl.kernel(
        body,
        out_type=[jax.ShapeDtypeStruct((NC, N, D), jnp.float32),
                  jax.ShapeDtypeStruct((NC, N, D), jnp.float32)],
        mesh=_MESH,
        compiler_params=_SC_PARAMS,
        scratch_types=[
            pltpu.VMEM_SHARED((N, D), jnp.float32),
            pltpu.VMEM_SHARED((N, D), jnp.float32),
            pltpu.VMEM((NB, K), jnp.int32),
            pltpu.VMEM((NB, K), jnp.int32),
            pltpu.VMEM((2, K, D), jnp.float32),
            pltpu.SemaphoreType.DMA,
        ],
    )


KD = 80
NBD = EPT // KD


def _deg_body(src_hbm, dst_hbm, ones_hbm, zrow_hbm,
              out_hbm, src_out, dst_out, accum, sidx, didx, ones_v):
    c = lax.axis_index("c")
    s = lax.axis_index("s")
    tid = s * NC + c
    r0 = pl.multiple_of(s * RPT, 8)
    pltpu.sync_copy(zrow_hbm, accum.at[pl.ds(r0, RPT)])

    @pl.when(s == NS - 1)
    def _():
        pltpu.sync_copy(zrow_hbm.at[pl.ds(0, TAIL)],
                        accum.at[pl.ds(NS * RPT, TAIL)])

    pltpu.sync_copy(src_hbm.at[tid], sidx)
    pltpu.sync_copy(dst_hbm.at[tid], didx)
    pltpu.sync_copy(sidx, src_out.at[tid])
    pltpu.sync_copy(didx, dst_out.at[tid])
    pltpu.sync_copy(ones_hbm, ones_v)
    plsc.subcore_barrier()

    def step(i, carry):
        pltpu.sync_copy(ones_v, accum.at[didx.at[i]], add=True)
        return carry

    lax.fori_loop(0, NBD, step, 0)
    plsc.subcore_barrier()
    pltpu.sync_copy(accum.at[pl.ds(r0, RPT)],
                    out_hbm.at[c, pl.ds(r0, RPT)])

    @pl.when(s == NS - 1)
    def _():
        pltpu.sync_copy(accum.at[pl.ds(NS * RPT, TAIL)],
                        out_hbm.at[c, pl.ds(NS * RPT, TAIL)])


_deg_call = pl.kernel(
    _deg_body,
    out_type=[jax.ShapeDtypeStruct((NC, N, 16), jnp.float32),
              jax.ShapeDtypeStruct((NW, NBD, KD), jnp.int32),
              jax.ShapeDtypeStruct((NW, NBD, KD), jnp.int32)],
    mesh=_MESH,
    compiler_params=_SC_PARAMS,
    scratch_types=[
        pltpu.VMEM_SHARED((N, 16), jnp.float32),
        pltpu.VMEM((NBD, KD), jnp.int32),
        pltpu.VMEM((NBD, KD), jnp.int32),
        pltpu.VMEM((KD, 16), jnp.float32),
    ],
)

_spmm64 = _make_spmm(64, 80)
_spmm16 = _make_spmm(16, 80)
_spmm64x2 = _make_spmm2(64, 80)


_B = 10000
_G = N // _B


def _mm1_body(d0_ref, d1_ref, x_ref, w_ref, dis_ref, oa_ref, ob_ref):
    dis = lax.rsqrt(d0_ref[0, :, 0:1] + d1_ref[0, :, 0:1] + 1.0)
    dis_ref[...] = dis
    hp = jnp.dot(x_ref[...], w_ref[...],
                 preferred_element_type=jnp.float32) * dis
    oa_ref[...] = hp[:, 0:64]
    ob_ref[...] = hp[:, 64:128]


def _mid1_body(sa0_ref, sa1_ref, sb0_ref, sb1_ref, ha_ref, hb_ref,
               dis_ref, b_ref, w_ref, o_ref):
    dis = dis_ref[...]
    ta = dis * (sa0_ref[0] + sa1_ref[0] + ha_ref[...]) + b_ref[:, 0:64]
    tb = dis * (sb0_ref[0] + sb1_ref[0] + hb_ref[...]) + b_ref[:, 64:128]
    t = jnp.concatenate([ta, tb], axis=1)
    t = jnp.where(t > 0, t, jnp.exp(t) - 1.0)
    o_ref[...] = jnp.dot(t, w_ref[...],
                         preferred_element_type=jnp.float32) * dis


def _mid_body(s0_ref, s1_ref, hp_ref, dis_ref, b_ref, w_ref, o_ref):
    dis = dis_ref[...]
    t = dis * (s0_ref[0] + s1_ref[0] + hp_ref[...]) + b_ref[...]
    t = jnp.where(t > 0, t, jnp.exp(t) - 1.0)
    o_ref[...] = jnp.dot(t, w_ref[...],
                         preferred_element_type=jnp.float32) * dis


def _final_body(s0_ref, s1_ref, hp_ref, dis_ref, b_ref, o_ref):
    acc = s0_ref[0, :, 0:1] + s1_ref[0, :, 0:1] + hp_ref[:, 0:1]
    o_ref[...] = dis_ref[...] * acc + b_ref[...]


def _row_spec(d):
    return pl.BlockSpec((_B, d), lambda i: (i, 0))


def _part_spec(d, core):
    return pl.BlockSpec((1, _B, d), lambda i, c=core: (c, i, 0))


def _full_spec(r, c):
    return pl.BlockSpec((r, c), lambda i: (0, 0))


def _mm1(dp, x, w):
    return pl.pallas_call(
        _mm1_body,
        grid=(_G,),
        in_specs=[_part_spec(16, 0), _part_spec(16, 1), _row_spec(128),
                  _full_spec(128, 128)],
        out_specs=[_row_spec(1), _row_spec(64), _row_spec(64)],
        out_shape=[jax.ShapeDtypeStruct((N, 1), jnp.float32),
                   jax.ShapeDtypeStruct((N, 64), jnp.float32),
                   jax.ShapeDtypeStruct((N, 64), jnp.float32)],
    )(dp, dp, x, w)


def _mid1(sa, sb, ha, hb, dis, b, w):
    return pl.pallas_call(
        _mid1_body,
        grid=(_G,),
        in_specs=[_part_spec(64, 0), _part_spec(64, 1),
                  _part_spec(64, 0), _part_spec(64, 1),
                  _row_spec(64), _row_spec(64), _row_spec(1),
                  _full_spec(1, 128), _full_spec(128, 64)],
        out_specs=_row_spec(64),
        out_shape=jax.ShapeDtypeStruct((N, 64), jnp.float32),
    )(sa, sa, sb, sb, ha, hb, dis, b, w)


def _mid(sp, hp, dis, b, w):
    din, dout = w.shape
    return pl.pallas_call(
        _mid_body,
        grid=(_G,),
        in_specs=[_part_spec(din, 0), _part_spec(din, 1), _row_spec(din),
                  _row_spec(1), _full_spec(1, din), _full_spec(din, dout)],
        out_specs=_row_spec(dout),
        out_shape=jax.ShapeDtypeStruct((N, dout), jnp.float32),
    )(sp, sp, hp, dis, b, w)


def _final(sp, hp, dis, b):
    return pl.pallas_call(
        _final_body,
        grid=(_G,),
        in_specs=[_part_spec(16, 0), _part_spec(16, 1), _row_spec(16),
                  _row_spec(1), _full_spec(1, 1)],
        out_specs=_row_spec(1),
        out_shape=jax.ShapeDtypeStruct((N, 1), jnp.float32),
    )(sp, sp, hp, dis, b)


def kernel(x, edge_index, W1, b1, W2, b2, W3, b3):
    ei = edge_index.astype(jnp.int32)
    src80 = ei[0].reshape(NW, EPT // 80, 80)
    dst80 = ei[1].reshape(NW, EPT // 80, 80)
    z64 = jnp.zeros((RPT, 64), jnp.float32)
    z16 = jnp.zeros((RPT, 16), jnp.float32)
    ones16 = jnp.ones((KD, 16), jnp.float32)

    degp, src_sc, dst_sc = _deg_call(src80, dst80, ones16, z16)
    dis, h1a, h1b = _mm1(degp, x, W1)
    s1a, s1b = _spmm64x2(h1a, h1b, src_sc, dst_sc, z64)
    h2p = _mid1(s1a, s1b, h1a, h1b, dis, b1.reshape(1, 128), W2)
    s2 = _spmm64(h2p, src_sc, dst_sc, z64)
    W3p = jnp.pad(W3, ((0, 0), (0, 15)))
    h3p = _mid(s2, h2p, dis, b2.reshape(1, 64), W3p)
    s3 = _spmm16(h3p, src_sc, dst_sc, z16)
    return _final(s3, h3p, dis, b3.reshape(1, 1))


# speedup vs baseline: 1.0262x; 1.0262x over previous
import functools

import jax
import jax.numpy as jnp
from jax import lax
from jax.experimental import pallas as pl
from jax.experimental.pallas import tpu as pltpu
from jax.experimental.pallas import tpu_sc as plsc

N = 10000
E = 320000
NC = 2
NS = 16
NW = NC * NS
EPT = E // NW
RPT = 624
TAIL = N - NS * RPT

_MESH = plsc.VectorSubcoreMesh(core_axis_name="c", subcore_axis_name="s")
_SC_PARAMS = pltpu.CompilerParams(use_tc_tiling_on_sc=False)


def _make_spmm(D, K):
    NB = EPT // K

    def body(h_hbm, src_hbm, dst_hbm, zrow_hbm, out_hbm,
             accum, htab, sidx, didx, rows, gsem):
        c = lax.axis_index("c")
        s = lax.axis_index("s")
        tid = s * NC + c
        r0 = pl.multiple_of(s * RPT, 8)
        pltpu.sync_copy(zrow_hbm, accum.at[pl.ds(r0, RPT)])
        pltpu.sync_copy(h_hbm.at[pl.ds(r0, RPT)], htab.at[pl.ds(r0, RPT)])

        @pl.when(s == NS - 1)
        def _():
            pltpu.sync_copy(zrow_hbm.at[pl.ds(0, TAIL)],
                            accum.at[pl.ds(NS * RPT, TAIL)])
            pltpu.sync_copy(h_hbm.at[pl.ds(NS * RPT, TAIL)],
                            htab.at[pl.ds(NS * RPT, TAIL)])

        pltpu.sync_copy(src_hbm.at[tid], sidx)
        pltpu.sync_copy(dst_hbm.at[tid], didx)
        plsc.subcore_barrier()

        def step(i, carry):
            b = lax.rem(i, 3)
            pltpu.make_async_copy(htab.at[sidx.at[i]], rows.at[b], gsem).wait()

            @pl.when(i + 2 < NB)
            def _():
                pltpu.async_copy(htab.at[sidx.at[i + 2]],
                                 rows.at[lax.rem(i + 2, 3)], gsem)

            pltpu.sync_copy(rows.at[b], accum.at[didx.at[i]], add=True)
            return carry

        pltpu.async_copy(htab.at[sidx.at[0]], rows.at[0], gsem)
        pltpu.async_copy(htab.at[sidx.at[1]], rows.at[1], gsem)
        lax.fori_loop(0, NB, step, 0)
        plsc.subcore_barrier()
        pltpu.sync_copy(accum.at[pl.ds(r0, RPT)],
                        out_hbm.at[c, pl.ds(r0, RPT)])

        @pl.when(s == NS - 1)
        def _():
            pltpu.sync_copy(accum.at[pl.ds(NS * RPT, TAIL)],
                            out_hbm.at[c, pl.ds(NS * RPT, TAIL)])

    return pl.kernel(
        body,
        out_type=jax.ShapeDtypeStruct((NC, N, D), jnp.float32),
        mesh=_MESH,
        compiler_params=_SC_PARAMS,
        scratch_types=[
            pltpu.VMEM_SHARED((N, D), jnp.float32),
            pltpu.VMEM_SHARED((N, D), jnp.float32),
            pltpu.VMEM((NB, K), jnp.int32),
            pltpu.VMEM((NB, K), jnp.int32),
            pltpu.VMEM((3, K, D), jnp.float32),
            pltpu.SemaphoreType.DMA,
        ],
    )


def _make_spmm2(D, K):
    NB = EPT // K

    def body(ha_hbm, hb_hbm, src_hbm, dst_hbm, zrow_hbm, outa_hbm, outb_hbm,
             accum, htab, sidx, didx, rows, gsem):
        c = lax.axis_index("c")
        s = lax.axis_index("s")
        tid = s * NC + c
        r0 = pl.multiple_of(s * RPT, 8)
        pltpu.sync_copy(src_hbm.at[tid], sidx)
        pltpu.sync_copy(dst_hbm.at[tid], didx)

        def run_pass(h_hbm, out_hbm):
            pltpu.sync_copy(zrow_hbm, accum.at[pl.ds(r0, RPT)])
            pltpu.sync_copy(h_hbm.at[pl.ds(r0, RPT)], htab.at[pl.ds(r0, RPT)])

            @pl.when(s == NS - 1)
            def _():
                pltpu.sync_copy(zrow_hbm.at[pl.ds(0, TAIL)],
                                accum.at[pl.ds(NS * RPT, TAIL)])
                pltpu.sync_copy(h_hbm.at[pl.ds(NS * RPT, TAIL)],
                                htab.at[pl.ds(NS * RPT, TAIL)])

            plsc.subcore_barrier()

            def step(i, carry):
                b = lax.rem(i, 3)
                pltpu.make_async_copy(htab.at[sidx.at[i]], rows.at[b],
                                      gsem).wait()

                @pl.when(i + 2 < NB)
                def _():
                    pltpu.async_copy(htab.at[sidx.at[i + 2]],
                                     rows.at[lax.rem(i + 2, 3)], gsem)

                pltpu.sync_copy(rows.at[b], accum.at[didx.at[i]], add=True)
                return carry

            pltpu.async_copy(htab.at[sidx.at[0]], rows.at[0], gsem)
            pltpu.async_copy(htab.at[sidx.at[1]], rows.at[1], gsem)
            lax.fori_loop(0, NB, step, 0)
            plsc.subcore_barrier()
            pltpu.sync_copy(accum.at[pl.ds(r0, RPT)],
                            out_hbm.at[c, pl.ds(r0, RPT)])

            @pl.when(s == NS - 1)
            def _():
                pltpu.sync_copy(accum.at[pl.ds(NS * RPT, TAIL)],
                                out_hbm.at[c, pl.ds(NS * RPT, TAIL)])

        run_pass(ha_hbm, outa_hbm)
        run_pass(hb_hbm, outb_hbm)

    return pl.kernel(
        body,
        out_type=[jax.ShapeDtypeStruct((NC, N, D), jnp.float32),
                  jax.ShapeDtypeStruct((NC, N, D), jnp.float32)],
        mesh=_MESH,
        compiler_params=_SC_PARAMS,
        scratch_types=[
            pltpu.VMEM_SHARED((N, D), jnp.float32),
            pltpu.VMEM_SHARED((N, D), jnp.float32),
            pltpu.VMEM((NB, K), jnp.int32),
            pltpu.VMEM((NB, K), jnp.int32),
            pltpu.VMEM((3, K, D), jnp.float32),
            pltpu.SemaphoreType.DMA,
        ],
    )


KD = 80
NBD = EPT // KD


def _deg_body(src_hbm, dst_hbm, ones_hbm, zrow_hbm,
              out_hbm, src_out, dst_out, accum, sidx, didx, ones_v):
    c = lax.axis_index("c")
    s = lax.axis_index("s")
    tid = s * NC + c
    r0 = pl.multiple_of(s * RPT, 8)
    pltpu.sync_copy(zrow_hbm, accum.at[pl.ds(r0, RPT)])

    @pl.when(s == NS - 1)
    def _():
        pltpu.sync_copy(zrow_hbm.at[pl.ds(0, TAIL)],
                        accum.at[pl.ds(NS * RPT, TAIL)])

    pltpu.sync_copy(src_hbm.at[tid], sidx)
    pltpu.sync_copy(dst_hbm.at[tid], didx)
    pltpu.sync_copy(sidx, src_out.at[tid])
    pltpu.sync_copy(didx, dst_out.at[tid])
    pltpu.sync_copy(ones_hbm, ones_v)
    plsc.subcore_barrier()

    def step(i, carry):
        pltpu.sync_copy(ones_v, accum.at[didx.at[i]], add=True)
        return carry

    lax.fori_loop(0, NBD, step, 0)
    plsc.subcore_barrier()
    pltpu.sync_copy(accum.at[pl.ds(r0, RPT)],
                    out_hbm.at[c, pl.ds(r0, RPT)])

    @pl.when(s == NS - 1)
    def _():
        pltpu.sync_copy(accum.at[pl.ds(NS * RPT, TAIL)],
                        out_hbm.at[c, pl.ds(NS * RPT, TAIL)])


_deg_call = pl.kernel(
    _deg_body,
    out_type=[jax.ShapeDtypeStruct((NC, N, 16), jnp.float32),
              jax.ShapeDtypeStruct((NW, NBD, KD), jnp.int32),
              jax.ShapeDtypeStruct((NW, NBD, KD), jnp.int32)],
    mesh=_MESH,
    compiler_params=_SC_PARAMS,
    scratch_types=[
        pltpu.VMEM_SHARED((N, 16), jnp.float32),
        pltpu.VMEM((NBD, KD), jnp.int32),
        pltpu.VMEM((NBD, KD), jnp.int32),
        pltpu.VMEM((KD, 16), jnp.float32),
    ],
)

_spmm64 = _make_spmm(64, 80)
_spmm16 = _make_spmm(16, 80)
_spmm64x2 = _make_spmm2(64, 80)


_B = 2000
_G = N // _B


def _mm1_body(d0_ref, d1_ref, x_ref, w_ref, dis_ref, oa_ref, ob_ref):
    dis = lax.rsqrt(d0_ref[0, :, 0:1] + d1_ref[0, :, 0:1] + 1.0)
    dis_ref[...] = dis
    hp = jnp.dot(x_ref[...], w_ref[...],
                 preferred_element_type=jnp.float32) * dis
    oa_ref[...] = hp[:, 0:64]
    ob_ref[...] = hp[:, 64:128]


def _mid1_body(sa0_ref, sa1_ref, sb0_ref, sb1_ref, ha_ref, hb_ref,
               dis_ref, b_ref, w_ref, o_ref):
    dis = dis_ref[...]
    ta = dis * (sa0_ref[0] + sa1_ref[0] + ha_ref[...]) + b_ref[:, 0:64]
    tb = dis * (sb0_ref[0] + sb1_ref[0] + hb_ref[...]) + b_ref[:, 64:128]
    t = jnp.concatenate([ta, tb], axis=1)
    t = jnp.where(t > 0, t, jnp.exp(t) - 1.0)
    o_ref[...] = jnp.dot(t, w_ref[...],
                         preferred_element_type=jnp.float32) * dis


def _mid_body(s0_ref, s1_ref, hp_ref, dis_ref, b_ref, w_ref, o_ref):
    dis = dis_ref[...]
    t = dis * (s0_ref[0] + s1_ref[0] + hp_ref[...]) + b_ref[...]
    t = jnp.where(t > 0, t, jnp.exp(t) - 1.0)
    o_ref[...] = jnp.dot(t, w_ref[...],
                         preferred_element_type=jnp.float32) * dis


def _final_body(s0_ref, s1_ref, hp_ref, dis_ref, b_ref, o_ref):
    acc = s0_ref[0, :, 0:1] + s1_ref[0, :, 0:1] + hp_ref[:, 0:1]
    o_ref[...] = dis_ref[...] * acc + b_ref[...]


def _row_spec(d):
    return pl.BlockSpec((_B, d), lambda i: (i, 0))


def _part_spec(d, core):
    return pl.BlockSpec((1, _B, d), lambda i, c=core: (c, i, 0))


def _full_spec(r, c):
    return pl.BlockSpec((r, c), lambda i: (0, 0))


def _mm1(dp, x, w):
    return pl.pallas_call(
        _mm1_body,
        grid=(_G,),
        in_specs=[_part_spec(16, 0), _part_spec(16, 1), _row_spec(128),
                  _full_spec(128, 128)],
        out_specs=[_row_spec(1), _row_spec(64), _row_spec(64)],
        out_shape=[jax.ShapeDtypeStruct((N, 1), jnp.float32),
                   jax.ShapeDtypeStruct((N, 64), jnp.float32),
                   jax.ShapeDtypeStruct((N, 64), jnp.float32)],
    )(dp, dp, x, w)


def _mid1(sa, sb, ha, hb, dis, b, w):
    return pl.pallas_call(
        _mid1_body,
        grid=(_G,),
        in_specs=[_part_spec(64, 0), _part_spec(64, 1),
                  _part_spec(64, 0), _part_spec(64, 1),
                  _row_spec(64), _row_spec(64), _row_spec(1),
                  _full_spec(1, 128), _full_spec(128, 64)],
        out_specs=_row_spec(64),
        out_shape=jax.ShapeDtypeStruct((N, 64), jnp.float32),
    )(sa, sa, sb, sb, ha, hb, dis, b, w)


def _mid(sp, hp, dis, b, w):
    din, dout = w.shape
    return pl.pallas_call(
        _mid_body,
        grid=(_G,),
        in_specs=[_part_spec(din, 0), _part_spec(din, 1), _row_spec(din),
                  _row_spec(1), _full_spec(1, din), _full_spec(din, dout)],
        out_specs=_row_spec(dout),
        out_shape=jax.ShapeDtypeStruct((N, dout), jnp.float32),
    )(sp, sp, hp, dis, b, w)


def _final(sp, hp, dis, b):
    return pl.pallas_call(
        _final_body,
        grid=(_G,),
        in_specs=[_part_spec(16, 0), _part_spec(16, 1), _row_spec(16),
                  _row_spec(1), _full_spec(1, 1)],
        out_specs=_row_spec(1),
        out_shape=jax.ShapeDtypeStruct((N, 1), jnp.float32),
    )(sp, sp, hp, dis, b)


def kernel(x, edge_index, W1, b1, W2, b2, W3, b3):
    ei = edge_index.astype(jnp.int32)
    src80 = ei[0].reshape(NW, EPT // 80, 80)
    dst80 = ei[1].reshape(NW, EPT // 80, 80)
    z64 = jnp.zeros((RPT, 64), jnp.float32)
    z16 = jnp.zeros((RPT, 16), jnp.float32)
    ones16 = jnp.ones((KD, 16), jnp.float32)

    degp, src_sc, dst_sc = _deg_call(src80, dst80, ones16, z16)
    dis, h1a, h1b = _mm1(degp, x, W1)
    s1a, s1b = _spmm64x2(h1a, h1b, src_sc, dst_sc, z64)
    h2p = _mid1(s1a, s1b, h1a, h1b, dis, b1.reshape(1, 128), W2)
    s2 = _spmm64(h2p, src_sc, dst_sc, z64)
    W3p = jnp.pad(W3, ((0, 0), (0, 15)))
    h3p = _mid(s2, h2p, dis, b2.reshape(1, 64), W3p)
    s3 = _spmm16(h3p, src_sc, dst_sc, z16)
    return _final(s3, h3p, dis, b3.reshape(1, 1))


# speedup vs baseline: 1.0269x; 1.0007x over previous
import jax
import jax.numpy as jnp
from jax import lax
from jax.experimental import pallas as pl
from jax.experimental.pallas import tpu as pltpu
from jax.experimental.pallas import tpu_sc as plsc

N = 10000
E = 320000
NC = 2
NS = 16
NW = NC * NS
EPT = E // NW
RPT = 624
TAIL = N - NS * RPT

_MESH = plsc.VectorSubcoreMesh(core_axis_name="c", subcore_axis_name="s")
_SC_PARAMS = pltpu.CompilerParams(use_tc_tiling_on_sc=False)


def _make_spmm(D, K):
    NB = EPT // K

    def body(h_hbm, src_hbm, dst_hbm, zrow_hbm, out_hbm,
             accum, htab, sidx, didx, rows, gsem):
        c = lax.axis_index("c")
        s = lax.axis_index("s")
        tid = s * NC + c
        r0 = pl.multiple_of(s * RPT, 8)
        pltpu.sync_copy(zrow_hbm, accum.at[pl.ds(r0, RPT)])
        pltpu.sync_copy(h_hbm.at[pl.ds(r0, RPT)], htab.at[pl.ds(r0, RPT)])

        @pl.when(s == NS - 1)
        def _():
            pltpu.sync_copy(zrow_hbm.at[pl.ds(0, TAIL)],
                            accum.at[pl.ds(NS * RPT, TAIL)])
            pltpu.sync_copy(h_hbm.at[pl.ds(NS * RPT, TAIL)],
                            htab.at[pl.ds(NS * RPT, TAIL)])

        pltpu.sync_copy(src_hbm.at[tid], sidx)
        pltpu.sync_copy(dst_hbm.at[tid], didx)
        plsc.subcore_barrier()

        def step(i, carry):
            b = lax.rem(i, 3)
            pltpu.make_async_copy(htab.at[sidx.at[i]], rows.at[b], gsem).wait()

            @pl.when(i + 2 < NB)
            def _():
                pltpu.async_copy(htab.at[sidx.at[i + 2]],
                                 rows.at[lax.rem(i + 2, 3)], gsem)

            pltpu.sync_copy(rows.at[b], accum.at[didx.at[i]], add=True)
            return carry

        pltpu.async_copy(htab.at[sidx.at[0]], rows.at[0], gsem)
        pltpu.async_copy(htab.at[sidx.at[1]], rows.at[1], gsem)
        lax.fori_loop(0, NB, step, 0)
        plsc.subcore_barrier()
        pltpu.sync_copy(accum.at[pl.ds(r0, RPT)],
                        out_hbm.at[c, pl.ds(r0, RPT)])

        @pl.when(s == NS - 1)
        def _():
            pltpu.sync_copy(accum.at[pl.ds(NS * RPT, TAIL)],
                            out_hbm.at[c, pl.ds(NS * RPT, TAIL)])

    return pl.kernel(
        body,
        out_type=jax.ShapeDtypeStruct((NC, N, D), jnp.float32),
        mesh=_MESH,
        compiler_params=_SC_PARAMS,
        scratch_types=[
            pltpu.VMEM_SHARED((N, D), jnp.float32),
            pltpu.VMEM_SHARED((N, D), jnp.float32),
            pltpu.VMEM((NB, K), jnp.int32),
            pltpu.VMEM((NB, K), jnp.int32),
            pltpu.VMEM((3, K, D), jnp.float32),
            pltpu.SemaphoreType.DMA,
        ],
    )


def _make_spmm2(D, K):
    NB = EPT // K

    def body(ha_hbm, hb_hbm, src_hbm, dst_hbm, zrow_hbm, outa_hbm, outb_hbm,
             accum, htab, sidx, didx, rows, gsem):
        c = lax.axis_index("c")
        s = lax.axis_index("s")
        tid = s * NC + c
        r0 = pl.multiple_of(s * RPT, 8)
        pltpu.sync_copy(src_hbm.at[tid], sidx)
        pltpu.sync_copy(dst_hbm.at[tid], didx)

        def run_pass(h_hbm, out_hbm):
            pltpu.sync_copy(zrow_hbm, accum.at[pl.ds(r0, RPT)])
            pltpu.sync_copy(h_hbm.at[pl.ds(r0, RPT)], htab.at[pl.ds(r0, RPT)])

            @pl.when(s == NS - 1)
            def _():
                pltpu.sync_copy(zrow_hbm.at[pl.ds(0, TAIL)],
                                accum.at[pl.ds(NS * RPT, TAIL)])
                pltpu.sync_copy(h_hbm.at[pl.ds(NS * RPT, TAIL)],
                                htab.at[pl.ds(NS * RPT, TAIL)])

            plsc.subcore_barrier()

            def step(i, carry):
                b = lax.rem(i, 3)
                pltpu.make_async_copy(htab.at[sidx.at[i]], rows.at[b],
                                      gsem).wait()

                @pl.when(i + 2 < NB)
                def _():
                    pltpu.async_copy(htab.at[sidx.at[i + 2]],
                                     rows.at[lax.rem(i + 2, 3)], gsem)

                pltpu.sync_copy(rows.at[b], accum.at[didx.at[i]], add=True)
                return carry

            pltpu.async_copy(htab.at[sidx.at[0]], rows.at[0], gsem)
            pltpu.async_copy(htab.at[sidx.at[1]], rows.at[1], gsem)
            lax.fori_loop(0, NB, step, 0)
            plsc.subcore_barrier()
            pltpu.sync_copy(accum.at[pl.ds(r0, RPT)],
                            out_hbm.at[c, pl.ds(r0, RPT)])

            @pl.when(s == NS - 1)
            def _():
                pltpu.sync_copy(accum.at[pl.ds(NS * RPT, TAIL)],
                                out_hbm.at[c, pl.ds(NS * RPT, TAIL)])

        run_pass(ha_hbm, outa_hbm)
        run_pass(hb_hbm, outb_hbm)

    return pl.kernel(
        body,
        out_type=[jax.ShapeDtypeStruct((NC, N, D), jnp.float32),
                  jax.ShapeDtypeStruct((NC, N, D), jnp.float32)],
        mesh=_MESH,
        compiler_params=_SC_PARAMS,
        scratch_types=[
            pltpu.VMEM_SHARED((N, D), jnp.float32),
            pltpu.VMEM_SHARED((N, D), jnp.float32),
            pltpu.VMEM((NB, K), jnp.int32),
            pltpu.VMEM((NB, K), jnp.int32),
            pltpu.VMEM((3, K, D), jnp.float32),
            pltpu.SemaphoreType.DMA,
        ],
    )


KD = 80
NBD = EPT // KD


def _deg_body(src_hbm, dst_hbm, ones_hbm, zrow_hbm,
              out_hbm, src_out, dst_out, accum, sidx, didx, ones_v):
    c = lax.axis_index("c")
    s = lax.axis_index("s")
    tid = s * NC + c
    r0 = pl.multiple_of(s * RPT, 8)
    pltpu.sync_copy(zrow_hbm, accum.at[pl.ds(r0, RPT)])

    @pl.when(s == NS - 1)
    def _():
        pltpu.sync_copy(zrow_hbm.at[pl.ds(0, TAIL)],
                        accum.at[pl.ds(NS * RPT, TAIL)])

    pltpu.sync_copy(src_hbm.at[tid], sidx)
    pltpu.sync_copy(dst_hbm.at[tid], didx)
    pltpu.sync_copy(sidx, src_out.at[tid])
    pltpu.sync_copy(didx, dst_out.at[tid])
    pltpu.sync_copy(ones_hbm, ones_v)
    plsc.subcore_barrier()

    def step(i, carry):
        pltpu.sync_copy(ones_v, accum.at[didx.at[i]], add=True)
        return carry

    lax.fori_loop(0, NBD, step, 0)
    plsc.subcore_barrier()
    pltpu.sync_copy(accum.at[pl.ds(r0, RPT)],
                    out_hbm.at[c, pl.ds(r0, RPT)])

    @pl.when(s == NS - 1)
    def _():
        pltpu.sync_copy(accum.at[pl.ds(NS * RPT, TAIL)],
                        out_hbm.at[c, pl.ds(NS * RPT, TAIL)])


_deg_call = pl.kernel(
    _deg_body,
    out_type=[jax.ShapeDtypeStruct((NC, N, 16), jnp.float32),
              jax.ShapeDtypeStruct((NW, NBD, KD), jnp.int32),
              jax.ShapeDtypeStruct((NW, NBD, KD), jnp.int32)],
    mesh=_MESH,
    compiler_params=_SC_PARAMS,
    scratch_types=[
        pltpu.VMEM_SHARED((N, 16), jnp.float32),
        pltpu.VMEM((NBD, KD), jnp.int32),
        pltpu.VMEM((NBD, KD), jnp.int32),
        pltpu.VMEM((KD, 16), jnp.float32),
    ],
)

_spmm64 = _make_spmm(64, 80)
_spmm16 = _make_spmm(16, 80)
_spmm64x2 = _make_spmm2(64, 80)


_B = 2000
_G = N // _B


def _mm1_body(d0_ref, d1_ref, x_ref, w_ref, dis_ref, oa_ref, ob_ref):
    dis = lax.rsqrt(d0_ref[0, :, 0:1] + d1_ref[0, :, 0:1] + 1.0)
    dis_ref[...] = dis
    hp = jnp.dot(x_ref[...], w_ref[...],
                 preferred_element_type=jnp.float32) * dis
    oa_ref[...] = hp[:, 0:64]
    ob_ref[...] = hp[:, 64:128]


def _mid1_body(sa0_ref, sa1_ref, sb0_ref, sb1_ref, ha_ref, hb_ref,
               dis_ref, b_ref, w_ref, o_ref):
    dis = dis_ref[...]
    ta = dis * (sa0_ref[0] + sa1_ref[0] + ha_ref[...]) + b_ref[:, 0:64]
    tb = dis * (sb0_ref[0] + sb1_ref[0] + hb_ref[...]) + b_ref[:, 64:128]
    t = jnp.concatenate([ta, tb], axis=1)
    t = jnp.where(t > 0, t, jnp.exp(t) - 1.0)
    o_ref[...] = jnp.dot(t, w_ref[...],
                         preferred_element_type=jnp.float32) * dis


def _mid_body(s0_ref, s1_ref, hp_ref, dis_ref, b_ref, w_ref, o_ref):
    dis = dis_ref[...]
    t = dis * (s0_ref[0] + s1_ref[0] + hp_ref[...]) + b_ref[...]
    t = jnp.where(t > 0, t, jnp.exp(t) - 1.0)
    o_ref[...] = jnp.dot(t, w_ref[...],
                         preferred_element_type=jnp.float32) * dis


def _final_body(s0_ref, s1_ref, hp_ref, dis_ref, b_ref, o_ref):
    acc = s0_ref[0, :, 0:1] + s1_ref[0, :, 0:1] + hp_ref[:, 0:1]
    o_ref[...] = dis_ref[...] * acc + b_ref[...]


def _row_spec(d):
    return pl.BlockSpec((_B, d), lambda i: (i, 0))


def _part_spec(d, core):
    return pl.BlockSpec((1, _B, d), lambda i, c=core: (c, i, 0))


def _full_spec(r, c):
    return pl.BlockSpec((r, c), lambda i: (0, 0))


def _mm1(dp, x, w):
    return pl.pallas_call(
        _mm1_body,
        grid=(_G,),
        in_specs=[_part_spec(16, 0), _part_spec(16, 1), _row_spec(128),
                  _full_spec(128, 128)],
        out_specs=[_row_spec(1), _row_spec(64), _row_spec(64)],
        out_shape=[jax.ShapeDtypeStruct((N, 1), jnp.float32),
                   jax.ShapeDtypeStruct((N, 64), jnp.float32),
                   jax.ShapeDtypeStruct((N, 64), jnp.float32)],
    )(dp, dp, x, w)


def _mid1(sa, sb, ha, hb, dis, b, w):
    return pl.pallas_call(
        _mid1_body,
        grid=(_G,),
        in_specs=[_part_spec(64, 0), _part_spec(64, 1),
                  _part_spec(64, 0), _part_spec(64, 1),
                  _row_spec(64), _row_spec(64), _row_spec(1),
                  _full_spec(1, 128), _full_spec(128, 64)],
        out_specs=_row_spec(64),
        out_shape=jax.ShapeDtypeStruct((N, 64), jnp.float32),
    )(sa, sa, sb, sb, ha, hb, dis, b, w)


def _mid(sp, hp, dis, b, w):
    din, dout = w.shape
    return pl.pallas_call(
        _mid_body,
        grid=(_G,),
        in_specs=[_part_spec(din, 0), _part_spec(din, 1), _row_spec(din),
                  _row_spec(1), _full_spec(1, din), _full_spec(din, dout)],
        out_specs=_row_spec(dout),
        out_shape=jax.ShapeDtypeStruct((N, dout), jnp.float32),
    )(sp, sp, hp, dis, b, w)


def _final(sp, hp, dis, b):
    return pl.pallas_call(
        _final_body,
        grid=(_G,),
        in_specs=[_part_spec(16, 0), _part_spec(16, 1), _row_spec(16),
                  _row_spec(1), _full_spec(1, 1)],
        out_specs=_row_spec(1),
        out_shape=jax.ShapeDtypeStruct((N, 1), jnp.float32),
    )(sp, sp, hp, dis, b)


def kernel(x, edge_index, W1, b1, W2, b2, W3, b3):
    ei = edge_index.astype(jnp.int32)
    src80 = ei[0].reshape(NW, EPT // 80, 80)
    dst80 = ei[1].reshape(NW, EPT // 80, 80)
    z64 = jnp.zeros((RPT, 64), jnp.float32)
    z16 = jnp.zeros((RPT, 16), jnp.float32)
    ones16 = jnp.ones((KD, 16), jnp.float32)

    degp, src_sc, dst_sc = _deg_call(src80, dst80, ones16, z16)
    dis, h1a, h1b = _mm1(degp, x, W1)
    s1a, s1b = _spmm64x2(h1a, h1b, src_sc, dst_sc, z64)
    h2p = _mid1(s1a, s1b, h1a, h1b, dis, b1.reshape(1, 128), W2)
    s2 = _spmm64(h2p, src_sc, dst_sc, z64)
    W3p = jnp.pad(W3, ((0, 0), (0, 15)))
    h3p = _mid(s2, h2p, dis, b2.reshape(1, 64), W3p)
    s3 = _spmm16(h3p, src_sc, dst_sc, z16)
    return _final(s3, h3p, dis, b3.reshape(1, 1))
